# baseline (speedup 1.0000x reference)
import functools

import jax
import jax.numpy as jnp
from jax import lax
from jax.experimental import pallas as pl
from jax.experimental.pallas import tpu as pltpu
from jax.experimental.pallas import tpu_sc as plsc

P_EPS = 1e-06
D = 128
B = 320000
NW = 32
CH = 80
NCH = 126
PW = CH * NCH
BP = NW * PW
LANES = 16

_mesh = plsc.VectorSubcoreMesh(core_axis_name="c", subcore_axis_name="s")


def _issue_gathers(z_hbm, si_v, di_v, c, sbuf, dbuf, sem):
    off = pl.multiple_of(c * CH, 8)
    pltpu.async_copy(z_hbm.at[si_v.at[pl.ds(off, CH)]], sbuf, sem)
    pltpu.async_copy(z_hbm.at[di_v.at[pl.ds(off, CH)]], dbuf, sem)


def _wait_gathers(z_hbm, sbuf, dbuf, sem):
    pltpu.make_async_copy(z_hbm.at[pl.ds(0, CH)], sbuf, sem).wait()
    pltpu.make_async_copy(z_hbm.at[pl.ds(0, CH)], dbuf, sem).wait()


_GATHER_DNUMS = lax.GatherDimensionNumbers(
    offset_dims=(), collapsed_slice_dims=(0,), start_index_map=(0,))


def _take16(x, idx):
    return lax.gather(x, idx[:, None], _GATHER_DNUMS, slice_sizes=(1,),
                      mode=lax.GatherScatterMode.PROMISE_IN_BOUNDS)


def _transpose_reduce(vecs, lane_ids):
    for m in (8, 4, 2, 1):
        mask = lax.bitwise_and(lane_ids, m) == 0
        perm = lax.bitwise_xor(lane_ids, m)
        half = len(vecs) // 2
        vecs = [
            jnp.where(mask, vecs[j], vecs[j + half])
            + _take16(jnp.where(mask, vecs[j + half], vecs[j]), perm)
            for j in range(half)
        ]
    return vecs[0]


def _compute_chunk(sbuf, dbuf, out_v, c):
    lane_ids = lax.iota(jnp.int32, LANES)

    def group_body(g, carry):
        row0 = g * LANES
        accs = []
        for l in range(LANES):
            acc = None
            for k in range(D // LANES):
                s = sbuf[row0 + l, pl.ds(k * LANES, LANES)]
                t = dbuf[row0 + l, pl.ds(k * LANES, LANES)]
                d = s - t + P_EPS
                acc = d * d if acc is None else acc + d * d
            accs.append(acc)
        out_v[pl.ds(c * CH + row0, LANES)] = _transpose_reduce(accs, lane_ids)
        return carry

    lax.fori_loop(0, CH // LANES, group_body, 0)


@functools.partial(
    pl.kernel,
    mesh=_mesh,
    out_type=jax.ShapeDtypeStruct((BP,), jnp.float32),
    scratch_types=[
        pltpu.VMEM((PW,), jnp.int32),
        pltpu.VMEM((PW,), jnp.int32),
        pltpu.VMEM((CH, D), jnp.float32),
        pltpu.VMEM((CH, D), jnp.float32),
        pltpu.VMEM((CH, D), jnp.float32),
        pltpu.VMEM((CH, D), jnp.float32),
        pltpu.VMEM((PW,), jnp.float32),
        pltpu.SemaphoreType.DMA,
        pltpu.SemaphoreType.DMA,
    ],
)
def _pnorm_sc(z_hbm, si_hbm, di_hbm, out_hbm,
              si_v, di_v, sa, da, sb, db, out_v, sem_a, sem_b):
    wid = lax.axis_index("s") * 2 + lax.axis_index("c")
    base = pl.multiple_of(wid * PW, 8)

    pltpu.sync_copy(si_hbm.at[pl.ds(base, PW)], si_v)
    pltpu.sync_copy(di_hbm.at[pl.ds(base, PW)], di_v)

    _issue_gathers(z_hbm, si_v, di_v, 0, sa, da, sem_a)

    def chunk_pair(j, carry):
        c0 = 2 * j
        _issue_gathers(z_hbm, si_v, di_v, c0 + 1, sb, db, sem_b)
        _wait_gathers(z_hbm, sa, da, sem_a)
        _compute_chunk(sa, da, out_v, c0)

        @pl.when(j < NCH // 2 - 1)
        def _():
            _issue_gathers(z_hbm, si_v, di_v, c0 + 2, sa, da, sem_a)

        _wait_gathers(z_hbm, sb, db, sem_b)
        _compute_chunk(sb, db, out_v, c0 + 1)
        return carry

    lax.fori_loop(0, NCH // 2, chunk_pair, 0)

    def pp_body(i, carry):
        x = out_v[pl.ds(i * LANES, LANES)]
        bits = lax.bitcast_convert_type(x, jnp.int32)
        y = lax.bitcast_convert_type(
            jnp.int32(0x5F3759DF) - (bits >> 1), jnp.float32)
        for _ in range(3):
            y = y * (1.5 - 0.5 * x * y * y)
        v = x * y
        out_v[pl.ds(i * LANES, LANES)] = 1.0 / (1.0 + jnp.exp(-v))
        return carry

    lax.fori_loop(0, PW // LANES, pp_body, 0)

    pltpu.sync_copy(out_v, out_hbm.at[pl.ds(base, PW)])


def kernel(z, edge_index):
    ei = edge_index.astype(jnp.int32)
    pad = jnp.zeros((BP - B,), jnp.int32)
    si = jnp.concatenate([ei[0], pad])
    di = jnp.concatenate([ei[1], pad])
    out = _pnorm_sc(z, si, di)
    return out[:B]


# speedup vs baseline: 2.5310x; 2.5310x over previous
import functools

import jax
import jax.numpy as jnp
from jax import lax
from jax.experimental import pallas as pl
from jax.experimental.pallas import tpu as pltpu
from jax.experimental.pallas import tpu_sc as plsc

P_EPS = 1e-06
D = 128
N_ROWS = 10000
B = 320000
NW = 32
CH = 32
NCH = 316
PW = CH * NCH
BP = NW * PW
LANES = 16

_mesh = plsc.VectorSubcoreMesh(core_axis_name="c", subcore_axis_name="s")


def _issue_gathers(z_sh, si_v, di_v, c, sbuf, dbuf, sem):
    off = pl.multiple_of(c * CH, 8)
    pltpu.async_copy(z_sh.at[si_v.at[pl.ds(off, CH)]], sbuf, sem)
    pltpu.async_copy(z_sh.at[di_v.at[pl.ds(off, CH)]], dbuf, sem)


def _wait_gathers(z_sh, sbuf, dbuf, sem):
    pltpu.make_async_copy(z_sh.at[pl.ds(0, CH)], sbuf, sem).wait()
    pltpu.make_async_copy(z_sh.at[pl.ds(0, CH)], dbuf, sem).wait()


_GATHER_DNUMS = lax.GatherDimensionNumbers(
    offset_dims=(), collapsed_slice_dims=(0,), start_index_map=(0,))


def _take16(x, idx):
    return lax.gather(x, idx[:, None], _GATHER_DNUMS, slice_sizes=(1,),
                      mode=lax.GatherScatterMode.PROMISE_IN_BOUNDS)


def _transpose_reduce(vecs, lane_ids):
    for m in (8, 4, 2, 1):
        mask = lax.bitwise_and(lane_ids, m) == 0
        perm = lax.bitwise_xor(lane_ids, m)
        half = len(vecs) // 2
        vecs = [
            jnp.where(mask, vecs[j], vecs[j + half])
            + _take16(jnp.where(mask, vecs[j + half], vecs[j]), perm)
            for j in range(half)
        ]
    return vecs[0]


def _compute_chunk(sbuf, dbuf, out_v, c):
    lane_ids = lax.iota(jnp.int32, LANES)

    def group_body(g, carry):
        row0 = g * LANES
        accs = []
        for l in range(LANES):
            acc = None
            for k in range(D // LANES):
                s = sbuf[row0 + l, pl.ds(k * LANES, LANES)]
                t = dbuf[row0 + l, pl.ds(k * LANES, LANES)]
                d = s - t + P_EPS
                acc = d * d if acc is None else acc + d * d
            accs.append(acc)
        out_v[pl.ds(c * CH + row0, LANES)] = _transpose_reduce(accs, lane_ids)
        return carry

    lax.fori_loop(0, CH // LANES, group_body, 0)


@functools.partial(
    pl.kernel,
    mesh=_mesh,
    out_type=jax.ShapeDtypeStruct((BP,), jnp.float32),
    scratch_types=[
        pltpu.VMEM((PW,), jnp.int32),
        pltpu.VMEM((PW,), jnp.int32),
        pltpu.VMEM((CH, D), jnp.float32),
        pltpu.VMEM((CH, D), jnp.float32),
        pltpu.VMEM((CH, D), jnp.float32),
        pltpu.VMEM((CH, D), jnp.float32),
        pltpu.VMEM((PW,), jnp.float32),
        pltpu.VMEM_SHARED((N_ROWS, D), jnp.float32),
        pltpu.SemaphoreType.DMA,
        pltpu.SemaphoreType.DMA,
    ],
)
def _pnorm_sc(z_hbm, si_hbm, di_hbm, out_hbm,
              si_v, di_v, sa, da, sb, db, out_v, z_sh, sem_a, sem_b):
    sid = lax.axis_index("s")
    wid = sid * 2 + lax.axis_index("c")
    base = pl.multiple_of(wid * PW, 8)

    zrow = pl.multiple_of(sid * 624, 8)

    @pl.when(sid < 15)
    def _():
        pltpu.sync_copy(z_hbm.at[pl.ds(zrow, 624)], z_sh.at[pl.ds(zrow, 624)])

    @pl.when(sid == 15)
    def _():
        pltpu.sync_copy(z_hbm.at[pl.ds(9360, 640)], z_sh.at[pl.ds(9360, 640)])

    pltpu.sync_copy(si_hbm.at[pl.ds(base, PW)], si_v)
    pltpu.sync_copy(di_hbm.at[pl.ds(base, PW)], di_v)
    plsc.subcore_barrier()

    _issue_gathers(z_sh, si_v, di_v, 0, sa, da, sem_a)

    def chunk_pair(j, carry):
        c0 = 2 * j
        _issue_gathers(z_sh, si_v, di_v, c0 + 1, sb, db, sem_b)
        _wait_gathers(z_sh, sa, da, sem_a)
        _compute_chunk(sa, da, out_v, c0)

        @pl.when(j < NCH // 2 - 1)
        def _():
            _issue_gathers(z_sh, si_v, di_v, c0 + 2, sa, da, sem_a)

        _wait_gathers(z_sh, sb, db, sem_b)
        _compute_chunk(sb, db, out_v, c0 + 1)
        return carry

    lax.fori_loop(0, NCH // 2, chunk_pair, 0)

    def pp_body(i, carry):
        x = out_v[pl.ds(i * LANES, LANES)]
        bits = lax.bitcast_convert_type(x, jnp.int32)
        y = lax.bitcast_convert_type(
            jnp.int32(0x5F3759DF) - (bits >> 1), jnp.float32)
        for _ in range(3):
            y = y * (1.5 - 0.5 * x * y * y)
        v = x * y
        out_v[pl.ds(i * LANES, LANES)] = 1.0 / (1.0 + jnp.exp(-v))
        return carry

    lax.fori_loop(0, PW // LANES, pp_body, 0)

    pltpu.sync_copy(out_v, out_hbm.at[pl.ds(base, PW)])


def kernel(z, edge_index):
    ei = edge_index.astype(jnp.int32)
    pad = jnp.zeros((BP - B,), jnp.int32)
    si = jnp.concatenate([ei[0], pad])
    di = jnp.concatenate([ei[1], pad])
    out = _pnorm_sc(z, si, di)
    return out[:B]
